# baseline (speedup 1.0000x reference)
import functools

import jax
import jax.numpy as jnp
from jax import lax
from jax.experimental import pallas as pl
from jax.experimental.pallas import tpu as pltpu
from jax.experimental.pallas import tpu_sc as plsc

B = 1
T = 2048
D = 768
E = 8
TOPK = 2
H = 1536

TM = 256
NTILES = 24
NPAD = TM * NTILES
NA = T * TOPK

_NC = 2
_NS = 16
_NW = _NC * _NS


def _routing_body(x_ref, rw_ref, rb_ref,
                  idx_ref, wn_ref, fe_ref, pe_ref, lb_ref, zl_ref, ent_ref):
    h = x_ref[...]
    logits = jnp.dot(h, rw_ref[...], preferred_element_type=jnp.float32)
    logits = logits + rb_ref[...]
    m = jnp.max(logits, axis=-1, keepdims=True)
    ex = jnp.exp(logits - m)
    s = jnp.sum(ex, axis=-1, keepdims=True)
    probs = ex / s
    z = m[:, 0] + jnp.log(s[:, 0])

    iota = jax.lax.broadcasted_iota(jnp.int32, (T, E), 1)
    m1 = jnp.max(probs, axis=-1, keepdims=True)
    idx1 = jnp.min(jnp.where(probs == m1, iota, E), axis=-1, keepdims=True)
    p2 = jnp.where(iota == idx1, -jnp.inf, probs)
    m2 = jnp.max(p2, axis=-1, keepdims=True)
    idx2 = jnp.min(jnp.where(p2 == m2, iota, E), axis=-1, keepdims=True)

    wsum = m1 + m2 + 1e-9
    oh1 = (iota == idx1).astype(jnp.float32)
    oh2 = (iota == idx2).astype(jnp.float32)
    idx_ref[...] = jnp.concatenate([idx1, idx2], axis=1)
    wn_ref[...] = jnp.concatenate([m1 / wsum, m2 / wsum], axis=1)

    fe = jnp.mean(oh1 + oh2, axis=0)[None, :]
    pe = jnp.mean(probs, axis=0)[None, :]
    fe_ref[...] = fe
    pe_ref[...] = pe
    lb_ref[...] = jnp.reshape(-E * jnp.sum(fe * pe), (1, 1))
    zl_ref[...] = jnp.reshape(jnp.mean(z * z), (1, 1))
    ent_ref[...] = jnp.reshape(
        jnp.mean(-jnp.sum(probs * jnp.log(probs + 1e-10), axis=-1)), (1, 1))


def _routing(h, router_w, router_b):
    outs = (
        jax.ShapeDtypeStruct((T, TOPK), jnp.int32),
        jax.ShapeDtypeStruct((T, TOPK), jnp.float32),
        jax.ShapeDtypeStruct((1, E), jnp.float32),
        jax.ShapeDtypeStruct((1, E), jnp.float32),
        jax.ShapeDtypeStruct((1, 1), jnp.float32),
        jax.ShapeDtypeStruct((1, 1), jnp.float32),
        jax.ShapeDtypeStruct((1, 1), jnp.float32),
    )
    return pl.pallas_call(
        _routing_body,
        out_shape=outs,
    )(h, router_w, router_b[None, :])


def _binning(idx):
    ee = jnp.arange(E, dtype=jnp.int32)[None, :]
    a0 = (idx[:, 0:1] == ee)
    a1 = (idx[:, 1:2] == ee)
    s = a0.astype(jnp.int32) + a1.astype(jnp.int32)
    cnt = s.sum(axis=0)
    tiles_pe = (cnt + TM - 1) // TM
    tile_hi = jnp.cumsum(tiles_pe)
    nused = tile_hi[-1].astype(jnp.int32)
    seg_start = (tile_hi - tiles_pe) * TM
    tix = jnp.arange(NTILES, dtype=jnp.int32)
    expert_of_tile = jnp.sum(
        (tix[:, None] >= tile_hi[None, :]).astype(jnp.int32), axis=1)
    expert_of_tile = jnp.minimum(expert_of_tile, E - 1)
    c = jnp.cumsum(s, axis=0) - s
    base = seg_start[None, :] + c
    dest0 = jnp.sum(jnp.where(a0, base, 0), axis=1).astype(jnp.int32)
    dest1 = jnp.sum(jnp.where(a1, base, 0), axis=1).astype(jnp.int32)
    eot = jnp.concatenate([expert_of_tile, nused[None]]).astype(jnp.int32)
    return dest0, dest1, eot


_TPW = T // _NW


def _sc_dispatch(dest0, dest1, h):
    mesh = plsc.VectorSubcoreMesh(core_axis_name="c", subcore_axis_name="s")

    @functools.partial(
        pl.kernel, mesh=mesh,
        out_type=jax.ShapeDtypeStruct((NPAD, D), jnp.float32),
        scratch_types=[
            pltpu.VMEM((_TPW,), jnp.int32),
            pltpu.VMEM((_TPW,), jnp.int32),
            pltpu.VMEM((_TPW, D), jnp.float32),
            pltpu.SemaphoreType.DMA,
            pltpu.SemaphoreType.DMA,
        ],
    )
    def k(d0_hbm, d1_hbm, x_hbm, out_hbm, i0, i1, xv, sem0, sem1):
        wid = lax.axis_index("s") * _NC + lax.axis_index("c")
        tbase = wid * _TPW
        pltpu.sync_copy(d0_hbm.at[pl.ds(tbase, _TPW)], i0)
        pltpu.sync_copy(d1_hbm.at[pl.ds(tbase, _TPW)], i1)
        pltpu.sync_copy(x_hbm.at[pl.ds(tbase, _TPW)], xv)
        cp0 = pltpu.async_copy(xv, out_hbm.at[i0], sem0)
        cp1 = pltpu.async_copy(xv, out_hbm.at[i1], sem1)
        cp0.wait()
        cp1.wait()

    return k(dest0, dest1, h)


def _sc_combine(gl, outs):
    per_w = NA // _NW
    chunk = 64
    nch = per_w // chunk
    mesh = plsc.VectorSubcoreMesh(core_axis_name="c", subcore_axis_name="s")

    @functools.partial(
        pl.kernel, mesh=mesh,
        out_type=jax.ShapeDtypeStruct((NA, D), jnp.float32),
        scratch_types=[
            pltpu.VMEM((per_w,), jnp.int32),
            pltpu.VMEM((chunk, D), jnp.float32),
            pltpu.VMEM((chunk, D), jnp.float32),
            pltpu.SemaphoreType.DMA,
            pltpu.SemaphoreType.DMA,
        ],
    )
    def k(gl_hbm, src_hbm, out_hbm, idx_v, buf0, buf1, sem0, sem1):
        wid = lax.axis_index("s") * _NC + lax.axis_index("c")
        base = wid * per_w
        pltpu.sync_copy(gl_hbm.at[pl.ds(base, per_w)], idx_v)
        bufs = (buf0, buf1)
        sems = (sem0, sem1)
        cps = []
        for c in range(min(2, nch)):
            cps.append(pltpu.async_copy(
                src_hbm.at[idx_v.at[pl.ds(c * chunk, chunk)]],
                bufs[c % 2], sems[c % 2]))
        for c in range(nch):
            cps[c].wait()
            pltpu.sync_copy(bufs[c % 2],
                            out_hbm.at[pl.ds(base + c * chunk, chunk)])
            nxt = c + 2
            if nxt < nch:
                cps.append(pltpu.async_copy(
                    src_hbm.at[idx_v.at[pl.ds(nxt * chunk, chunk)]],
                    bufs[nxt % 2], sems[nxt % 2]))

    return k(gl, outs)


def _ffn_body(eot_ref, xs_ref, w1_ref, b1_ref, w2_ref, b2_ref, out_ref):
    i = pl.program_id(0)

    @pl.when(i < eot_ref[NTILES])
    def _compute():
        xb = xs_ref[...].astype(jnp.bfloat16)
        hid = jnp.dot(xb, w1_ref[0], preferred_element_type=jnp.float32)
        hid = hid + b1_ref[0]
        hid = hid * 0.5 * (1.0 + jax.lax.erf(hid * 0.7071067811865476))
        out = jnp.dot(hid.astype(jnp.bfloat16), w2_ref[0],
                      preferred_element_type=jnp.float32)
        out_ref[...] = out + b2_ref[0]


def _ffn_grouped(xs, eot, w1, b1, w2, b2):
    grid_spec = pltpu.PrefetchScalarGridSpec(
        num_scalar_prefetch=1,
        grid=(NTILES,),
        in_specs=[
            pl.BlockSpec((TM, D), lambda i, eot: (i, 0)),
            pl.BlockSpec((1, D, H), lambda i, eot: (eot[i], 0, 0)),
            pl.BlockSpec((1, 1, H), lambda i, eot: (eot[i], 0, 0)),
            pl.BlockSpec((1, H, D), lambda i, eot: (eot[i], 0, 0)),
            pl.BlockSpec((1, 1, D), lambda i, eot: (eot[i], 0, 0)),
        ],
        out_specs=pl.BlockSpec((TM, D), lambda i, eot: (i, 0)),
    )
    return pl.pallas_call(
        _ffn_body,
        grid_spec=grid_spec,
        out_shape=jax.ShapeDtypeStruct((NPAD, D), jnp.float32),
    )(eot, xs, w1.astype(jnp.bfloat16), b1[:, None, :],
      w2.astype(jnp.bfloat16), b2[:, None, :])


def _finalize_body(r0_ref, r1_ref, wn_ref, y_ref):
    y_ref[...] = (r0_ref[...] * wn_ref[:, 0:1]
                  + r1_ref[...] * wn_ref[:, 1:2])


def _finalize(rows, wn):
    return pl.pallas_call(
        _finalize_body,
        grid=(1,),
        in_specs=[
            pl.BlockSpec((T, D), lambda i: (0, 0)),
            pl.BlockSpec((T, D), lambda i: (1, 0)),
            pl.BlockSpec((T, TOPK), lambda i: (0, 0)),
        ],
        out_specs=pl.BlockSpec((T, D), lambda i: (0, 0)),
        out_shape=jax.ShapeDtypeStruct((T, D), jnp.float32),
    )(rows, rows, wn)


def kernel(x, router_w, router_b, w1, b1, w2, b2):
    h = x.reshape(T, D)
    idx, wn, fe, pe, lb, zl, ent = _routing(h, router_w, router_b)
    dest0, dest1, eot = _binning(idx)
    xs = _sc_dispatch(dest0, dest1, h)
    outs = _ffn_grouped(xs, eot, w1, b1, w2, b2)
    rows = _sc_combine(jnp.concatenate([dest0, dest1]), outs)
    y = _finalize(rows, wn)
    return (y.reshape(B, T, D), lb[0, 0], zl[0, 0], ent[0, 0],
            fe[0], pe[0])


# speedup vs baseline: 1.3270x; 1.3270x over previous
import functools

import jax
import jax.numpy as jnp
from jax import lax
from jax.experimental import pallas as pl
from jax.experimental.pallas import tpu as pltpu
from jax.experimental.pallas import tpu_sc as plsc

B = 1
T = 2048
D = 768
E = 8
TOPK = 2
H = 1536

TM = 512
NTILES = 15
NPAD = TM * NTILES
NA = T * TOPK

_NC = 2
_NS = 16
_NW = _NC * _NS


def _routing_body(x_ref, rw_ref, rb_ref,
                  idx_ref, wn_ref, fe_ref, pe_ref, lb_ref, zl_ref, ent_ref):
    h = x_ref[...]
    logits = jnp.dot(h, rw_ref[...], preferred_element_type=jnp.float32)
    logits = logits + rb_ref[...]
    m = jnp.max(logits, axis=-1, keepdims=True)
    ex = jnp.exp(logits - m)
    s = jnp.sum(ex, axis=-1, keepdims=True)
    probs = ex / s
    z = m[:, 0] + jnp.log(s[:, 0])

    iota = jax.lax.broadcasted_iota(jnp.int32, (T, E), 1)
    m1 = jnp.max(probs, axis=-1, keepdims=True)
    idx1 = jnp.min(jnp.where(probs == m1, iota, E), axis=-1, keepdims=True)
    p2 = jnp.where(iota == idx1, -jnp.inf, probs)
    m2 = jnp.max(p2, axis=-1, keepdims=True)
    idx2 = jnp.min(jnp.where(p2 == m2, iota, E), axis=-1, keepdims=True)

    wsum = m1 + m2 + 1e-9
    oh1 = (iota == idx1).astype(jnp.float32)
    oh2 = (iota == idx2).astype(jnp.float32)
    idx_ref[...] = jnp.concatenate([idx1, idx2], axis=1)
    wn_ref[...] = jnp.concatenate([m1 / wsum, m2 / wsum], axis=1)

    fe = jnp.mean(oh1 + oh2, axis=0)[None, :]
    pe = jnp.mean(probs, axis=0)[None, :]
    fe_ref[...] = fe
    pe_ref[...] = pe
    lb_ref[...] = jnp.reshape(-E * jnp.sum(fe * pe), (1, 1))
    zl_ref[...] = jnp.reshape(jnp.mean(z * z), (1, 1))
    ent_ref[...] = jnp.reshape(
        jnp.mean(-jnp.sum(probs * jnp.log(probs + 1e-10), axis=-1)), (1, 1))


def _routing(h, router_w, router_b):
    outs = (
        jax.ShapeDtypeStruct((T, TOPK), jnp.int32),
        jax.ShapeDtypeStruct((T, TOPK), jnp.float32),
        jax.ShapeDtypeStruct((1, E), jnp.float32),
        jax.ShapeDtypeStruct((1, E), jnp.float32),
        jax.ShapeDtypeStruct((1, 1), jnp.float32),
        jax.ShapeDtypeStruct((1, 1), jnp.float32),
        jax.ShapeDtypeStruct((1, 1), jnp.float32),
    )
    return pl.pallas_call(
        _routing_body,
        out_shape=outs,
    )(h, router_w, router_b[None, :])


def _binning(idx):
    ee = jnp.arange(E, dtype=jnp.int32)[None, :]
    a0 = (idx[:, 0:1] == ee)
    a1 = (idx[:, 1:2] == ee)
    s = a0.astype(jnp.int32) + a1.astype(jnp.int32)
    cnt = s.sum(axis=0)
    tiles_pe = (cnt + TM - 1) // TM
    tile_hi = jnp.cumsum(tiles_pe)
    nused = tile_hi[-1].astype(jnp.int32)
    seg_start = (tile_hi - tiles_pe) * TM
    tix = jnp.arange(NTILES, dtype=jnp.int32)
    expert_of_tile = jnp.sum(
        (tix[:, None] >= tile_hi[None, :]).astype(jnp.int32), axis=1)
    expert_of_tile = jnp.minimum(expert_of_tile, E - 1)
    c = jnp.cumsum(s, axis=0) - s
    base = seg_start[None, :] + c
    dest0 = jnp.sum(jnp.where(a0, base, 0), axis=1).astype(jnp.int32)
    dest1 = jnp.sum(jnp.where(a1, base, 0), axis=1).astype(jnp.int32)
    eot = jnp.concatenate([expert_of_tile, nused[None]]).astype(jnp.int32)
    return dest0, dest1, eot


_TPW = T // _NW


def _sc_dispatch(dest0, dest1, h):
    mesh = plsc.VectorSubcoreMesh(core_axis_name="c", subcore_axis_name="s")

    @functools.partial(
        pl.kernel, mesh=mesh,
        out_type=jax.ShapeDtypeStruct((NPAD, D), jnp.float32),
        scratch_types=[
            pltpu.VMEM((_TPW,), jnp.int32),
            pltpu.VMEM((_TPW,), jnp.int32),
            pltpu.VMEM((_TPW, D), jnp.float32),
            pltpu.SemaphoreType.DMA,
            pltpu.SemaphoreType.DMA,
        ],
    )
    def k(d0_hbm, d1_hbm, x_hbm, out_hbm, i0, i1, xv, sem0, sem1):
        wid = lax.axis_index("s") * _NC + lax.axis_index("c")
        tbase = wid * _TPW
        pltpu.sync_copy(d0_hbm.at[pl.ds(tbase, _TPW)], i0)
        pltpu.sync_copy(d1_hbm.at[pl.ds(tbase, _TPW)], i1)
        pltpu.sync_copy(x_hbm.at[pl.ds(tbase, _TPW)], xv)
        cp0 = pltpu.async_copy(xv, out_hbm.at[i0], sem0)
        cp1 = pltpu.async_copy(xv, out_hbm.at[i1], sem1)
        cp0.wait()
        cp1.wait()

    return k(dest0, dest1, h)


def _sc_combine(gl, outs):
    per_w = NA // _NW
    chunk = 64
    nch = per_w // chunk
    mesh = plsc.VectorSubcoreMesh(core_axis_name="c", subcore_axis_name="s")

    @functools.partial(
        pl.kernel, mesh=mesh,
        out_type=jax.ShapeDtypeStruct((NA, D), jnp.float32),
        scratch_types=[
            pltpu.VMEM((per_w,), jnp.int32),
            pltpu.VMEM((chunk, D), jnp.float32),
            pltpu.VMEM((chunk, D), jnp.float32),
            pltpu.SemaphoreType.DMA,
            pltpu.SemaphoreType.DMA,
        ],
    )
    def k(gl_hbm, src_hbm, out_hbm, idx_v, buf0, buf1, sem0, sem1):
        wid = lax.axis_index("s") * _NC + lax.axis_index("c")
        base = wid * per_w
        pltpu.sync_copy(gl_hbm.at[pl.ds(base, per_w)], idx_v)
        bufs = (buf0, buf1)
        sems = (sem0, sem1)
        cps = []
        for c in range(min(2, nch)):
            cps.append(pltpu.async_copy(
                src_hbm.at[idx_v.at[pl.ds(c * chunk, chunk)]],
                bufs[c % 2], sems[c % 2]))
        for c in range(nch):
            cps[c].wait()
            pltpu.sync_copy(bufs[c % 2],
                            out_hbm.at[pl.ds(base + c * chunk, chunk)])
            nxt = c + 2
            if nxt < nch:
                cps.append(pltpu.async_copy(
                    src_hbm.at[idx_v.at[pl.ds(nxt * chunk, chunk)]],
                    bufs[nxt % 2], sems[nxt % 2]))

    return k(gl, outs)


def _ffn_body(eot_ref, xs_ref, w1_ref, b1_ref, w2_ref, b2_ref, out_ref):
    i = pl.program_id(0)

    @pl.when(i < eot_ref[NTILES])
    def _compute():
        hid = jnp.dot(xs_ref[...], w1_ref[0],
                      preferred_element_type=jnp.float32)
        hid = hid + b1_ref[0]
        hid = hid * 0.5 * (1.0 + jax.lax.erf(hid * 0.7071067811865476))
        out = jnp.dot(hid, w2_ref[0], preferred_element_type=jnp.float32)
        out_ref[...] = out + b2_ref[0]


def _ffn_grouped(xs, eot, w1, b1, w2, b2):
    grid_spec = pltpu.PrefetchScalarGridSpec(
        num_scalar_prefetch=1,
        grid=(NTILES,),
        in_specs=[
            pl.BlockSpec((TM, D), lambda i, eot: (i, 0)),
            pl.BlockSpec((1, D, H), lambda i, eot: (eot[i], 0, 0)),
            pl.BlockSpec((1, 1, H), lambda i, eot: (eot[i], 0, 0)),
            pl.BlockSpec((1, H, D), lambda i, eot: (eot[i], 0, 0)),
            pl.BlockSpec((1, 1, D), lambda i, eot: (eot[i], 0, 0)),
        ],
        out_specs=pl.BlockSpec((TM, D), lambda i, eot: (i, 0)),
    )
    return pl.pallas_call(
        _ffn_body,
        grid_spec=grid_spec,
        out_shape=jax.ShapeDtypeStruct((NPAD, D), jnp.float32),
    )(eot, xs, w1, b1[:, None, :], w2, b2[:, None, :])


def _finalize_body(r0_ref, r1_ref, wn_ref, y_ref):
    y_ref[...] = (r0_ref[...] * wn_ref[:, 0:1]
                  + r1_ref[...] * wn_ref[:, 1:2])


def _finalize(rows, wn):
    return pl.pallas_call(
        _finalize_body,
        grid=(1,),
        in_specs=[
            pl.BlockSpec((T, D), lambda i: (0, 0)),
            pl.BlockSpec((T, D), lambda i: (1, 0)),
            pl.BlockSpec((T, TOPK), lambda i: (0, 0)),
        ],
        out_specs=pl.BlockSpec((T, D), lambda i: (0, 0)),
        out_shape=jax.ShapeDtypeStruct((T, D), jnp.float32),
    )(rows, rows, wn)


def kernel(x, router_w, router_b, w1, b1, w2, b2):
    h = x.reshape(T, D)
    idx, wn, fe, pe, lb, zl, ent = _routing(h, router_w, router_b)
    dest0, dest1, eot = _binning(idx)
    xs = _sc_dispatch(dest0, dest1, h)
    outs = _ffn_grouped(xs, eot, w1, b1, w2, b2)
    rows = _sc_combine(jnp.concatenate([dest0, dest1]), outs)
    y = _finalize(rows, wn)
    return (y.reshape(B, T, D), lb[0, 0], zl[0, 0], ent[0, 0],
            fe[0], pe[0])


# speedup vs baseline: 1.3759x; 1.0368x over previous
import functools

import jax
import jax.numpy as jnp
from jax import lax
from jax.experimental import pallas as pl
from jax.experimental.pallas import tpu as pltpu
from jax.experimental.pallas import tpu_sc as plsc

B = 1
T = 2048
D = 768
E = 8
TOPK = 2
H = 1536

TM = 512
NTILES = 15
NPAD = TM * NTILES
NA = T * TOPK

_NC = 2
_NS = 16
_NW = _NC * _NS


def _routing_body(x_ref, rw_ref, rb_ref,
                  wn_ref, fe_ref, pe_ref, lb_ref, zl_ref, ent_ref,
                  dests_ref, pf_ref):
    h = x_ref[...]
    logits = jnp.dot(h, rw_ref[...], preferred_element_type=jnp.float32)
    logits = logits + rb_ref[...]
    m = jnp.max(logits, axis=-1, keepdims=True)
    ex = jnp.exp(logits - m)
    s = jnp.sum(ex, axis=-1, keepdims=True)
    probs = ex / s
    z = m[:, 0] + jnp.log(s[:, 0])

    iota = jax.lax.broadcasted_iota(jnp.int32, (T, E), 1)
    m1 = jnp.max(probs, axis=-1, keepdims=True)
    idx1 = jnp.min(jnp.where(probs == m1, iota, E), axis=-1, keepdims=True)
    p2 = jnp.where(iota == idx1, -jnp.inf, probs)
    m2 = jnp.max(p2, axis=-1, keepdims=True)
    idx2 = jnp.min(jnp.where(p2 == m2, iota, E), axis=-1, keepdims=True)

    wsum = m1 + m2 + 1e-9
    wn_ref[...] = jnp.concatenate([m1 / wsum, m2 / wsum], axis=1)

    pe = jnp.mean(probs, axis=0)[None, :]
    pe_ref[...] = pe
    zl_ref[...] = jnp.reshape(jnp.mean(z * z), (1, 1))
    ent_ref[...] = jnp.reshape(
        jnp.mean(-jnp.sum(probs * jnp.log(probs + 1e-10), axis=-1)), (1, 1))

    i1r = idx1.reshape(1, T)
    i2r = idx2.reshape(1, T)
    eiota = jax.lax.broadcasted_iota(jnp.int32, (E, T), 0)
    a0t = eiota == i1r
    a1t = eiota == i2r
    st = a0t.astype(jnp.float32) + a1t.astype(jnp.float32)
    c = st
    sh = 1
    while sh < T:
        c = c + jnp.concatenate(
            [jnp.zeros((E, sh), jnp.float32), c[:, :-sh]], axis=1)
        sh *= 2
    cnt = c[:, T - 1:T]
    tiles_pe = jnp.ceil(cnt * (1.0 / TM))
    tl = tiles_pe
    for sh2 in (1, 2, 4):
        tl = tl + jnp.concatenate(
            [jnp.zeros((sh2, 1), jnp.float32), tl[:-sh2]], axis=0)
    tile_lo = tl - tiles_pe
    baset = tile_lo * TM + (c - st)
    d0 = jnp.sum(jnp.where(a0t, baset, 0.0), axis=0, keepdims=True)
    d1 = jnp.sum(jnp.where(a1t, baset, 0.0), axis=0, keepdims=True)
    dests_ref[...] = jnp.concatenate([d0, d1], axis=0).astype(jnp.int32)

    fe = cnt.reshape(1, E) * (1.0 / T)
    fe_ref[...] = fe
    lb_ref[...] = jnp.reshape(-E * jnp.sum(fe * pe), (1, 1))
    pf_ref[...] = jnp.concatenate(
        [tile_lo.reshape(1, E), tiles_pe.reshape(1, E),
         jnp.sum(tiles_pe).reshape(1, 1)], axis=1).astype(jnp.int32)


def _routing(h, router_w, router_b):
    outs = (
        jax.ShapeDtypeStruct((T, TOPK), jnp.float32),
        jax.ShapeDtypeStruct((1, E), jnp.float32),
        jax.ShapeDtypeStruct((1, E), jnp.float32),
        jax.ShapeDtypeStruct((1, 1), jnp.float32),
        jax.ShapeDtypeStruct((1, 1), jnp.float32),
        jax.ShapeDtypeStruct((1, 1), jnp.float32),
        jax.ShapeDtypeStruct((TOPK, T), jnp.int32),
        jax.ShapeDtypeStruct((1, 2 * E + 1), jnp.int32),
    )
    return pl.pallas_call(
        _routing_body,
        out_shape=outs,
    )(h, router_w, router_b[None, :])


_TPW = T // _NW


def _sc_dispatch(dest0, dest1, h):
    mesh = plsc.VectorSubcoreMesh(core_axis_name="c", subcore_axis_name="s")

    @functools.partial(
        pl.kernel, mesh=mesh,
        out_type=jax.ShapeDtypeStruct((NPAD, D), jnp.float32),
        scratch_types=[
            pltpu.VMEM((_TPW,), jnp.int32),
            pltpu.VMEM((_TPW,), jnp.int32),
            pltpu.VMEM((_TPW, D), jnp.float32),
            pltpu.SemaphoreType.DMA,
            pltpu.SemaphoreType.DMA,
        ],
    )
    def k(d0_hbm, d1_hbm, x_hbm, out_hbm, i0, i1, xv, sem0, sem1):
        wid = lax.axis_index("s") * _NC + lax.axis_index("c")
        tbase = wid * _TPW
        pltpu.sync_copy(d0_hbm.at[pl.ds(tbase, _TPW)], i0)
        pltpu.sync_copy(d1_hbm.at[pl.ds(tbase, _TPW)], i1)
        pltpu.sync_copy(x_hbm.at[pl.ds(tbase, _TPW)], xv)
        cp0 = pltpu.async_copy(xv, out_hbm.at[i0], sem0)
        cp1 = pltpu.async_copy(xv, out_hbm.at[i1], sem1)
        cp0.wait()
        cp1.wait()

    return k(dest0, dest1, h)


def _sc_combine(gl, outs):
    per_w = NA // _NW
    chunk = 64
    nch = per_w // chunk
    mesh = plsc.VectorSubcoreMesh(core_axis_name="c", subcore_axis_name="s")

    @functools.partial(
        pl.kernel, mesh=mesh,
        out_type=jax.ShapeDtypeStruct((NA, D), jnp.float32),
        scratch_types=[
            pltpu.VMEM((per_w,), jnp.int32),
            pltpu.VMEM((chunk, D), jnp.float32),
            pltpu.VMEM((chunk, D), jnp.float32),
            pltpu.SemaphoreType.DMA,
            pltpu.SemaphoreType.DMA,
        ],
    )
    def k(gl_hbm, src_hbm, out_hbm, idx_v, buf0, buf1, sem0, sem1):
        wid = lax.axis_index("s") * _NC + lax.axis_index("c")
        base = wid * per_w
        pltpu.sync_copy(gl_hbm.at[pl.ds(base, per_w)], idx_v)
        bufs = (buf0, buf1)
        sems = (sem0, sem1)
        cps = []
        for c in range(min(2, nch)):
            cps.append(pltpu.async_copy(
                src_hbm.at[idx_v.at[pl.ds(c * chunk, chunk)]],
                bufs[c % 2], sems[c % 2]))
        for c in range(nch):
            cps[c].wait()
            pltpu.sync_copy(bufs[c % 2],
                            out_hbm.at[pl.ds(base + c * chunk, chunk)])
            nxt = c + 2
            if nxt < nch:
                cps.append(pltpu.async_copy(
                    src_hbm.at[idx_v.at[pl.ds(nxt * chunk, chunk)]],
                    bufs[nxt % 2], sems[nxt % 2]))

    return k(gl, outs)


def _ffn_body(pf_ref, xs_ref, w1_ref, b1_ref, w2_ref, b2_ref, out_ref):
    i = pl.program_id(0)

    @pl.when(i < pf_ref[2 * E])
    def _compute():
        hid = jnp.dot(xs_ref[...], w1_ref[0],
                      preferred_element_type=jnp.float32)
        hid = hid + b1_ref[0]
        hid = hid * 0.5 * (1.0 + jax.lax.erf(hid * 0.7071067811865476))
        out = jnp.dot(hid, w2_ref[0], preferred_element_type=jnp.float32)
        out_ref[...] = out + b2_ref[0]


def _eot(i, pf):
    e = jnp.int32(0)
    for j in range(E):
        e = e + (i >= pf[j] + pf[E + j]).astype(jnp.int32)
    return jnp.minimum(e, E - 1)


def _ffn_grouped(xs, pf, w1, b1, w2, b2):
    grid_spec = pltpu.PrefetchScalarGridSpec(
        num_scalar_prefetch=1,
        grid=(NTILES,),
        in_specs=[
            pl.BlockSpec((TM, D), lambda i, pf: (i, 0)),
            pl.BlockSpec((1, D, H), lambda i, pf: (_eot(i, pf), 0, 0)),
            pl.BlockSpec((1, 1, H), lambda i, pf: (_eot(i, pf), 0, 0)),
            pl.BlockSpec((1, H, D), lambda i, pf: (_eot(i, pf), 0, 0)),
            pl.BlockSpec((1, 1, D), lambda i, pf: (_eot(i, pf), 0, 0)),
        ],
        out_specs=pl.BlockSpec((TM, D), lambda i, pf: (i, 0)),
    )
    return pl.pallas_call(
        _ffn_body,
        grid_spec=grid_spec,
        out_shape=jax.ShapeDtypeStruct((NPAD, D), jnp.float32),
    )(pf, xs, w1, b1[:, None, :], w2, b2[:, None, :])


def _finalize_body(r0_ref, r1_ref, wn_ref, y_ref):
    y_ref[...] = (r0_ref[...] * wn_ref[:, 0:1]
                  + r1_ref[...] * wn_ref[:, 1:2])


def _finalize(rows, wn):
    return pl.pallas_call(
        _finalize_body,
        grid=(1,),
        in_specs=[
            pl.BlockSpec((T, D), lambda i: (0, 0)),
            pl.BlockSpec((T, D), lambda i: (1, 0)),
            pl.BlockSpec((T, TOPK), lambda i: (0, 0)),
        ],
        out_specs=pl.BlockSpec((T, D), lambda i: (0, 0)),
        out_shape=jax.ShapeDtypeStruct((T, D), jnp.float32),
    )(rows, rows, wn)


def kernel(x, router_w, router_b, w1, b1, w2, b2):
    h = x.reshape(T, D)
    wn, fe, pe, lb, zl, ent, dests, pf2 = _routing(h, router_w, router_b)
    pf = pf2.reshape(2 * E + 1)
    dest0 = dests[0]
    dest1 = dests[1]
    xs = _sc_dispatch(dest0, dest1, h)
    outs = _ffn_grouped(xs, pf, w1, b1, w2, b2)
    rows = _sc_combine(dests.reshape(NA), outs)
    y = _finalize(rows, wn)
    return (y.reshape(B, T, D), lb[0, 0], zl[0, 0], ent[0, 0],
            fe[0], pe[0])
